# baseline (speedup 1.0000x reference)
import jax
import jax.numpy as jnp
from jax.experimental import pallas as pl
from jax.experimental.pallas import tpu as pltpu

_B = 1024
_D = 4 * 64 * 64
_BB = 32


def _fwd_kernel(t_sref, sac_sref, somac_sref, x0_ref, noise_ref, xt_ref, tgt_ref):
    b = pl.program_id(0)
    rows = jax.lax.broadcasted_iota(jnp.int32, (_BB, 1), 0)
    sa = jnp.zeros((_BB, 1), jnp.float32)
    so = jnp.zeros((_BB, 1), jnp.float32)
    for i in range(_BB):
        ti = t_sref[b * _BB + i]
        sa = jnp.where(rows == i, sac_sref[ti], sa)
        so = jnp.where(rows == i, somac_sref[ti], so)
    x = x0_ref[...]
    n = noise_ref[...]
    xt_ref[...] = sa * x + so * n
    tgt_ref[...] = sa * n - so * x


def kernel(x0, t, noise, sqrt_alphas_cumprod, sqrt_one_minus_alphas_cumprod):
    x0r = x0.reshape(_B, _D)
    nr = noise.reshape(_B, _D)
    t32 = t.astype(jnp.int32)

    grid_spec = pltpu.PrefetchScalarGridSpec(
        num_scalar_prefetch=3,
        grid=(_B // _BB,),
        in_specs=[
            pl.BlockSpec((_BB, _D), lambda b, *_: (b, 0)),
            pl.BlockSpec((_BB, _D), lambda b, *_: (b, 0)),
        ],
        out_specs=[
            pl.BlockSpec((_BB, _D), lambda b, *_: (b, 0)),
            pl.BlockSpec((_BB, _D), lambda b, *_: (b, 0)),
        ],
    )
    xt, tgt = pl.pallas_call(
        _fwd_kernel,
        grid_spec=grid_spec,
        compiler_params=pltpu.CompilerParams(
            dimension_semantics=("parallel",),
        ),
        out_shape=[
            jax.ShapeDtypeStruct((_B, _D), jnp.float32),
            jax.ShapeDtypeStruct((_B, _D), jnp.float32),
        ],
    )(t32, sqrt_alphas_cumprod, sqrt_one_minus_alphas_cumprod, x0r, nr)
    return xt.reshape(x0.shape), tgt.reshape(x0.shape)


# speedup vs baseline: 1.0071x; 1.0071x over previous
import jax
import jax.numpy as jnp
from jax.experimental import pallas as pl
from jax.experimental.pallas import tpu as pltpu

_B = 1024
_D = 4 * 64 * 64
_BB = 64


def _fwd_kernel(t_sref, sac_sref, somac_sref, x0_ref, noise_ref, xt_ref, tgt_ref):
    b = pl.program_id(0)
    rows = jax.lax.broadcasted_iota(jnp.int32, (_BB, 1), 0)
    sa = jnp.zeros((_BB, 1), jnp.float32)
    so = jnp.zeros((_BB, 1), jnp.float32)
    for i in range(_BB):
        ti = t_sref[b * _BB + i]
        sa = jnp.where(rows == i, sac_sref[ti], sa)
        so = jnp.where(rows == i, somac_sref[ti], so)
    x = x0_ref[...]
    n = noise_ref[...]
    xt_ref[...] = sa * x + so * n
    tgt_ref[...] = sa * n - so * x


def kernel(x0, t, noise, sqrt_alphas_cumprod, sqrt_one_minus_alphas_cumprod):
    x0r = x0.reshape(_B, _D)
    nr = noise.reshape(_B, _D)
    t32 = t.astype(jnp.int32)

    grid_spec = pltpu.PrefetchScalarGridSpec(
        num_scalar_prefetch=3,
        grid=(_B // _BB,),
        in_specs=[
            pl.BlockSpec((_BB, _D), lambda b, *_: (b, 0)),
            pl.BlockSpec((_BB, _D), lambda b, *_: (b, 0)),
        ],
        out_specs=[
            pl.BlockSpec((_BB, _D), lambda b, *_: (b, 0)),
            pl.BlockSpec((_BB, _D), lambda b, *_: (b, 0)),
        ],
    )
    xt, tgt = pl.pallas_call(
        _fwd_kernel,
        grid_spec=grid_spec,
        compiler_params=pltpu.CompilerParams(
            dimension_semantics=("parallel",),
        ),
        out_shape=[
            jax.ShapeDtypeStruct((_B, _D), jnp.float32),
            jax.ShapeDtypeStruct((_B, _D), jnp.float32),
        ],
    )(t32, sqrt_alphas_cumprod, sqrt_one_minus_alphas_cumprod, x0r, nr)
    return xt.reshape(x0.shape), tgt.reshape(x0.shape)
